# Initial kernel scaffold; baseline (speedup 1.0000x reference)
import jax
import jax.numpy as jnp
from jax.experimental import pallas as pl


def kernel(x, W1, b1, W2, b2, Wg, bg):
    raise NotImplementedError("write your pallas kernel here")


# speedup vs baseline: 3.7444x; 3.7444x over previous
import jax
import jax.numpy as jnp
from jax.experimental import pallas as pl
from jax.experimental.pallas import tpu as pltpu

DIM = 768
FF = 4 * DIM
E = 8
S = 2048
TOPK = 2
TS = 256
LANES = 128


def _gate_kernel(x_ref, wg_ref, bg_ref, sel_ref, p_ref, aux_ref):
    x = x_ref[...]
    logits = jnp.dot(x, wg_ref[...], preferred_element_type=jnp.float32)
    logits = logits + bg_ref[...]
    col = jax.lax.broadcasted_iota(jnp.int32, logits.shape, 1)
    valid = col < E
    neg = jnp.float32(-jnp.inf)
    logits = jnp.where(valid, logits, neg)

    m = jnp.max(logits, axis=1, keepdims=True)
    ex = jnp.where(valid, jnp.exp(logits - m), 0.0)
    probs = ex / jnp.sum(ex, axis=1, keepdims=True)

    e0 = jnp.argmax(logits, axis=1)
    l0 = jnp.max(logits, axis=1)
    masked = jnp.where(col == e0[:, None], neg, logits)
    e1 = jnp.argmax(masked, axis=1)
    l1 = jnp.max(masked, axis=1)
    w0 = 1.0 / (1.0 + jnp.exp(l1 - l0))
    w1 = 1.0 - w0

    is0 = col == e0[:, None]
    is1 = col == e1[:, None]
    p_ref[...] = jnp.where(is0, w0[:, None], 0.0) + jnp.where(is1, w1[:, None], 0.0)

    count_mask = ((is0 | is1) & valid).astype(jnp.float32)
    me = jnp.mean(probs, axis=0)
    ce = jnp.mean(count_mask, axis=0)
    aux = E * jnp.sum(me * ce)
    aux_ref[...] = jnp.full((1, LANES), aux, dtype=jnp.float32)

    sel = jnp.where(col == 0, e0[:, None], jnp.where(col == 1, e1[:, None], 0))
    sel_ref[...] = sel.astype(jnp.int32)


def _expert_kernel(x_ref, w1_ref, b1_ref, w2_ref, b2_ref, p_ref, out_ref):
    e = pl.program_id(0)
    t = pl.program_id(1)
    xt = x_ref[...]
    h = jnp.dot(xt, w1_ref[0], preferred_element_type=jnp.float32) + b1_ref[0]
    h = 0.5 * h * (1.0 + jax.lax.erf(h * 0.7071067811865476))
    y = jnp.dot(h, w2_ref[0], preferred_element_type=jnp.float32) + b2_ref[0]
    lane = jax.lax.broadcasted_iota(jnp.int32, (TS, LANES), 1)
    w = jnp.sum(jnp.where(lane == e, p_ref[...], 0.0), axis=1, keepdims=True)
    contrib = y * w
    row = t * TS

    @pl.when(e == 0)
    def _():
        out_ref[pl.ds(row, TS), :] = contrib

    @pl.when(e > 0)
    def _():
        out_ref[pl.ds(row, TS), :] = out_ref[pl.ds(row, TS), :] + contrib


def kernel(x, W1, b1, W2, b2, Wg, bg):
    x2 = x.reshape(S, DIM)
    wg_p = jnp.pad(Wg, ((0, 0), (0, LANES - E)))
    bg_p = jnp.pad(bg, (0, LANES - E)).reshape(1, LANES)

    sel_pad, P, aux = pl.pallas_call(
        _gate_kernel,
        out_shape=[
            jax.ShapeDtypeStruct((S, LANES), jnp.int32),
            jax.ShapeDtypeStruct((S, LANES), jnp.float32),
            jax.ShapeDtypeStruct((1, LANES), jnp.float32),
        ],
    )(x2, wg_p, bg_p)

    b1r = b1.reshape(E, 1, FF)
    b2r = b2.reshape(E, 1, DIM)
    out = pl.pallas_call(
        _expert_kernel,
        grid=(E, S // TS),
        in_specs=[
            pl.BlockSpec((TS, DIM), lambda e, t: (t, 0)),
            pl.BlockSpec((1, DIM, FF), lambda e, t: (e, 0, 0)),
            pl.BlockSpec((1, 1, FF), lambda e, t: (e, 0, 0)),
            pl.BlockSpec((1, FF, DIM), lambda e, t: (e, 0, 0)),
            pl.BlockSpec((1, 1, DIM), lambda e, t: (e, 0, 0)),
            pl.BlockSpec((TS, LANES), lambda e, t: (t, 0)),
        ],
        out_specs=pl.BlockSpec((S, DIM), lambda e, t: (0, 0)),
        out_shape=jax.ShapeDtypeStruct((S, DIM), jnp.float32),
        compiler_params=pltpu.CompilerParams(
            dimension_semantics=("arbitrary", "arbitrary"),
        ),
    )(x2, W1, b1r, W2, b2r, P)

    output = out.reshape(1, S, DIM)
    selected = sel_pad[:, :TOPK].reshape(1, S, TOPK)
    aux_loss = aux[0, 0]
    return (output, selected, aux_loss)


# speedup vs baseline: 4.9783x; 1.3295x over previous
import jax
import jax.numpy as jnp
from jax import lax
from jax.experimental import pallas as pl
from jax.experimental.pallas import tpu as pltpu
from jax.experimental.pallas import tpu_sc as plsc

DIM = 768
FF = 4 * DIM
E = 8
S = 2048
P = 2 * S
TOPK = 2
LANES = 128
TS2 = 128
G = P // TS2 + E
ROWS = G * TS2
NG = 32
GS = P // NG
NW = 32
CTS = 256


def _gate_kernel(x_ref, wg_ref, bg_ref, sel_ref, aux_ref, w_ref, pos_ref,
                 texp_ref):
    x = x_ref[...]
    logits = jnp.dot(x, wg_ref[...], preferred_element_type=jnp.float32)
    logits = logits + bg_ref[...]
    col = lax.broadcasted_iota(jnp.int32, (S, LANES), 1)
    valid = col < E
    neg = jnp.float32(-jnp.inf)
    logits = jnp.where(valid, logits, neg)

    m = jnp.max(logits, axis=1, keepdims=True)
    ex = jnp.where(valid, jnp.exp(logits - m), 0.0)
    probs = ex / jnp.sum(ex, axis=1, keepdims=True)

    e0 = jnp.argmax(logits, axis=1)
    l0 = jnp.max(logits, axis=1)
    masked = jnp.where(col == e0[:, None], neg, logits)
    e1 = jnp.argmax(masked, axis=1)
    l1 = jnp.max(masked, axis=1)
    w0 = 1.0 / (1.0 + jnp.exp(l1 - l0))
    w1 = 1.0 - w0

    is0 = col == e0[:, None]
    is1 = col == e1[:, None]
    count_mask = ((is0 | is1) & valid).astype(jnp.float32)
    me = jnp.mean(probs, axis=0)
    ce = jnp.mean(count_mask, axis=0)
    aux_ref[...] = jnp.full((1, LANES), E * jnp.sum(me * ce), jnp.float32)
    sel = jnp.where(col == 0, e0[:, None], jnp.where(col == 1, e1[:, None], 0))
    sel_ref[...] = sel.astype(jnp.int32)

    e_pair = jnp.concatenate([e0[:, None], e1[:, None]], axis=0)
    pcol = lax.broadcasted_iota(jnp.int32, (P, LANES), 1)
    onehot = (pcol == e_pair).astype(jnp.float32)

    ti = lax.broadcasted_iota(jnp.int32, (GS, GS), 0)
    tj = lax.broadcasted_iota(jnp.int32, (GS, GS), 1)
    tril = (tj <= ti).astype(jnp.float32)
    gsum = jnp.concatenate(
        [jnp.sum(onehot[g * GS:(g + 1) * GS], axis=0, keepdims=True)
         for g in range(NG)], axis=0)
    gi = lax.broadcasted_iota(jnp.int32, (NG, NG), 0)
    gj = lax.broadcasted_iota(jnp.int32, (NG, NG), 1)
    gtril = (gj < gi).astype(jnp.float32)
    gpre = jnp.dot(gtril, gsum, preferred_element_type=jnp.float32)
    counts = gpre[NG - 1:NG, :] + gsum[NG - 1:NG, :]

    ntiles = jnp.floor((counts + (TS2 - 1)) * (1.0 / TS2))
    ei = lax.broadcasted_iota(jnp.int32, (LANES, LANES), 0)
    ej = lax.broadcasted_iota(jnp.int32, (LANES, LANES), 1)
    upper = (ei < ej).astype(jnp.float32)
    tbase = jnp.dot(ntiles, upper, preferred_element_type=jnp.float32)
    base_rows = tbase * TS2

    rank = jnp.concatenate(
        [gpre[g:g + 1, :] +
         jnp.dot(tril, onehot[g * GS:(g + 1) * GS],
                 preferred_element_type=jnp.float32)
         for g in range(NG)], axis=0)
    pos = jnp.sum(onehot * (base_rows + rank - 1.0), axis=1, keepdims=True)
    pos_ref[...] = jnp.broadcast_to(pos, (P, LANES)).astype(jnp.int32)
    wpair = jnp.concatenate([w0[:, None], w1[:, None]], axis=0)
    w_ref[...] = jnp.broadcast_to(wpair, (P, LANES))

    tend = tbase + ntiles
    grow = ei.astype(jnp.float32)
    hit = jnp.where((ej < E) & (grow >= tend), 1.0, 0.0)
    texp = jnp.minimum(jnp.sum(hit, axis=1, keepdims=True), E - 1)
    texp_ref[...] = jnp.broadcast_to(texp, (LANES, LANES)).astype(jnp.int32)


def _vmesh():
    return plsc.VectorSubcoreMesh(core_axis_name="c", subcore_axis_name="s")


def _sc_scatter_kernel(x_hbm, pos_hbm, o_hbm, idx_v, rows_v):
    c = lax.axis_index("c")
    s = lax.axis_index("s")
    wid = s * 2 + c
    n = P // NW
    base = wid * n
    tok = lax.rem(base, S)
    pltpu.sync_copy(pos_hbm.at[pl.ds(base, n)], idx_v)
    pltpu.sync_copy(x_hbm.at[pl.ds(tok, n)], rows_v)
    pltpu.sync_copy(rows_v, o_hbm.at[idx_v])


def _sc_scatter(x2, pos):
    k = pl.kernel(
        _sc_scatter_kernel,
        out_type=jax.ShapeDtypeStruct((ROWS, DIM), jnp.float32),
        mesh=_vmesh(),
        scratch_types=[
            pltpu.VMEM((P // NW,), jnp.int32),
            pltpu.VMEM((P // NW, DIM), jnp.float32),
        ],
    )
    return k(x2, pos)


def _sc_gather_kernel(y_hbm, pos_hbm, a_hbm, b_hbm, idx_v, rows_v):
    c = lax.axis_index("c")
    s = lax.axis_index("s")
    wid = s * 2 + c
    n = S // NW
    base = wid * n
    pltpu.sync_copy(pos_hbm.at[pl.ds(base, n)], idx_v)
    pltpu.sync_copy(y_hbm.at[idx_v], rows_v)
    pltpu.sync_copy(rows_v, a_hbm.at[pl.ds(base, n)])
    pltpu.sync_copy(pos_hbm.at[pl.ds(S + base, n)], idx_v)
    pltpu.sync_copy(y_hbm.at[idx_v], rows_v)
    pltpu.sync_copy(rows_v, b_hbm.at[pl.ds(base, n)])


def _sc_gather(y_sorted, pos):
    k = pl.kernel(
        _sc_gather_kernel,
        out_type=[
            jax.ShapeDtypeStruct((S, DIM), jnp.float32),
            jax.ShapeDtypeStruct((S, DIM), jnp.float32),
        ],
        mesh=_vmesh(),
        scratch_types=[
            pltpu.VMEM((S // NW,), jnp.int32),
            pltpu.VMEM((S // NW, DIM), jnp.float32),
        ],
    )
    return k(y_sorted, pos)


def _mlp_kernel(tmap_ref, x_ref, w1_ref, b1_ref, w2_ref, b2_ref, out_ref):
    h = jnp.dot(x_ref[...], w1_ref[0], preferred_element_type=jnp.float32)
    h = h + b1_ref[0]
    h = 0.5 * h * (1.0 + lax.erf(h * 0.7071067811865476))
    y = jnp.dot(h, w2_ref[0], preferred_element_type=jnp.float32)
    out_ref[...] = y + b2_ref[0]


def _grouped_mlp(tile_map, x_sorted, W1, b1, W2, b2):
    b1r = b1.reshape(E, 1, FF)
    b2r = b2.reshape(E, 1, DIM)
    grid_spec = pltpu.PrefetchScalarGridSpec(
        num_scalar_prefetch=1,
        grid=(G,),
        in_specs=[
            pl.BlockSpec((TS2, DIM), lambda g, m: (g, 0)),
            pl.BlockSpec((1, DIM, FF), lambda g, m: (m[g], 0, 0)),
            pl.BlockSpec((1, 1, FF), lambda g, m: (m[g], 0, 0)),
            pl.BlockSpec((1, FF, DIM), lambda g, m: (m[g], 0, 0)),
            pl.BlockSpec((1, 1, DIM), lambda g, m: (m[g], 0, 0)),
        ],
        out_specs=pl.BlockSpec((TS2, DIM), lambda g, m: (g, 0)),
    )
    return pl.pallas_call(
        _mlp_kernel,
        grid_spec=grid_spec,
        out_shape=jax.ShapeDtypeStruct((ROWS, DIM), jnp.float32),
        compiler_params=pltpu.CompilerParams(
            dimension_semantics=("arbitrary",),
        ),
    )(tile_map, x_sorted, W1, b1r, W2, b2r)


def _combine_kernel(a_ref, b_ref, wa_ref, wb_ref, out_ref):
    out_ref[...] = (a_ref[...] * wa_ref[:, 0:1] +
                    b_ref[...] * wb_ref[:, 0:1])


def _combine(a_rows, b_rows, w_big):
    return pl.pallas_call(
        _combine_kernel,
        grid=(S // CTS,),
        in_specs=[
            pl.BlockSpec((CTS, DIM), lambda t: (t, 0)),
            pl.BlockSpec((CTS, DIM), lambda t: (t, 0)),
            pl.BlockSpec((CTS, LANES), lambda t: (t, 0)),
            pl.BlockSpec((CTS, LANES), lambda t: (t + S // CTS, 0)),
        ],
        out_specs=pl.BlockSpec((CTS, DIM), lambda t: (t, 0)),
        out_shape=jax.ShapeDtypeStruct((S, DIM), jnp.float32),
    )(a_rows, b_rows, w_big, w_big)


def kernel(x, W1, b1, W2, b2, Wg, bg):
    x2 = x.reshape(S, DIM)
    wg_p = jnp.pad(Wg, ((0, 0), (0, LANES - E)))
    bg_p = jnp.pad(bg, (0, LANES - E)).reshape(1, LANES)

    sel_pad, aux, w_big, pos_big, texp_big = pl.pallas_call(
        _gate_kernel,
        out_shape=[
            jax.ShapeDtypeStruct((S, LANES), jnp.int32),
            jax.ShapeDtypeStruct((1, LANES), jnp.float32),
            jax.ShapeDtypeStruct((P, LANES), jnp.float32),
            jax.ShapeDtypeStruct((P, LANES), jnp.int32),
            jax.ShapeDtypeStruct((LANES, LANES), jnp.int32),
        ],
    )(x2, wg_p, bg_p)

    pos = pos_big[:, 0]
    tile_map = texp_big[:G, 0]

    x_sorted = _sc_scatter(x2, pos)
    y_sorted = _grouped_mlp(tile_map, x_sorted, W1, b1, W2, b2)
    a_rows, b_rows = _sc_gather(y_sorted, pos)
    out = _combine(a_rows, b_rows, w_big)

    output = out.reshape(1, S, DIM)
    selected = sel_pad[:, :TOPK].reshape(1, S, TOPK)
    aux_loss = aux[0, 0]
    return (output, selected, aux_loss)
